# baseline (speedup 1.0000x reference)
import functools

import jax
import jax.numpy as jnp
from jax import lax
from jax.experimental import pallas as pl
from jax.experimental.pallas import tpu as pltpu
from jax.experimental.pallas import tpu_sc as plsc

F = 16
LANES = 16
NC = 2
NS = 16
NW = NC * NS


def _round_up(v, m):
    return (v + m - 1) // m * m


def _largest_divisor_leq(n, cap):
    for d in range(min(cap, n), 0, -1):
        if n % d == 0:
            return d
    return 1


def _make_sc_layer(n_nodes, erows, in_dim, n_blocks, out_dim, ws, count_col):
    ibk = in_dim // n_blocks
    obk = out_dim // n_blocks
    rows_per_w = erows // NW
    ib = 8 * _largest_divisor_leq(rows_per_w // 8, 2)
    nblk = rows_per_w // ib
    acc_n = _round_up(n_nodes + 1, 2048)
    zps = acc_n // NS
    nz = zps // 128
    ob = 128
    no = zps // ob

    mesh = plsc.VectorSubcoreMesh(core_axis_name="c", subcore_axis_name="s")

    def body(h_hbm, src_hbm, dst_hbm, typ_hbm, wtab_hbm, out_hbm,
             wtab_v, srcb, dstb, typb, rows0, rows1, rows2, rows3,
             msg0, msg1, acc_sh, g0, g1, g2, g3, sem2, sem3):
        c = lax.axis_index("c")
        s = lax.axis_index("s")
        wid = s * NC + c

        pltpu.sync_copy(wtab_hbm, wtab_v)

        zero16 = jnp.zeros((LANES,), jnp.float32)
        for i in range(128):
            msg0[i, :] = zero16
            msg1[i, :] = zero16

        def zero_body(k, _):
            pltpu.sync_copy(msg0, acc_sh.at[pl.ds(s * zps + k * 128, 128)])
            return 0
        lax.fori_loop(0, nz, zero_body, 0)
        plsc.subcore_barrier()

        lanes = lax.iota(jnp.int32, LANES)
        ones16 = jnp.ones((LANES,), jnp.float32)

        row0 = wid * rows_per_w

        def compute_row(rbuf, mbuf, j):
            for g in range(128 // LANES):
                rowi = g * LANES + lanes
                tvec = typb[j, pl.ds(g * LANES, LANES)]
                hc = [plsc.load_gather(
                          rbuf, [rowi, jnp.full((LANES,), i, jnp.int32)])
                      for i in range(in_dim)]
                wc = [plsc.load_gather(
                          wtab_v, [tvec, jnp.full((LANES,), k, jnp.int32)])
                      for k in range(n_blocks * ibk * obk)]
                for o in range(out_dim):
                    b = o // obk
                    oo = o % obk
                    acc = hc[b * ibk] * wc[(b * ibk) * obk + oo]
                    for i in range(1, ibk):
                        acc = acc + hc[b * ibk + i] * wc[(b * ibk + i) * obk + oo]
                    plsc.store_scatter(
                        mbuf, [rowi, jnp.full((LANES,), o, jnp.int32)], acc)
                if count_col is not None:
                    plsc.store_scatter(
                        mbuf,
                        [rowi, jnp.full((LANES,), count_col, jnp.int32)],
                        ones16)

        def blk_body(blk, _):
            rbase = row0 + blk * ib
            pltpu.sync_copy(src_hbm.at[pl.ds(rbase, ib)], srcb)
            pltpu.sync_copy(dst_hbm.at[pl.ds(rbase, ib)], dstb)
            pltpu.sync_copy(typ_hbm.at[pl.ds(rbase, ib)], typb)

            rbufs = (rows0, rows1, rows2, rows3)
            gsems = (g0, g1, g2, g3)
            for p in range(3):
                pltpu.async_copy(h_hbm.at[srcb.at[p]], rbufs[p], gsems[p])

            def quad_body(jq, _):
                j0 = 4 * jq
                for u in range(4):
                    j = j0 + u
                    rbuf, gsem = rbufs[u], gsems[u]
                    mbuf, msem = (msg0, sem2) if u % 2 == 0 else (msg1, sem3)
                    @pl.when(j + 3 < ib)
                    def _():
                        pltpu.async_copy(h_hbm.at[srcb.at[j + 3]],
                                         rbufs[(u + 3) % 4],
                                         gsems[(u + 3) % 4])
                    pltpu.make_async_copy(h_hbm.at[srcb.at[j]], rbuf,
                                          gsem).wait()

                    @pl.when(j0 + u >= 2)
                    def _():
                        pltpu.make_async_copy(mbuf, acc_sh.at[dstb.at[j]],
                                              msem).wait()
                    compute_row(rbuf, mbuf, j)
                    pltpu.async_copy(mbuf, acc_sh.at[dstb.at[j]], msem,
                                     add=True)
                return 0
            lax.fori_loop(0, ib // 4, quad_body, 0)
            pltpu.make_async_copy(msg0, acc_sh.at[dstb.at[0]], sem2).wait()
            pltpu.make_async_copy(msg1, acc_sh.at[dstb.at[1]], sem3).wait()
            return 0
        lax.fori_loop(0, nblk, blk_body, 0)
        plsc.subcore_barrier()

        def out_body(k, _):
            r = s * zps + k * ob
            pltpu.sync_copy(acc_sh.at[pl.ds(r, ob)], rows0)
            pltpu.sync_copy(rows0, out_hbm.at[c, pl.ds(r, ob)])
            return 0
        lax.fori_loop(0, no, out_body, 0)

    def run(h16, src_rows, dst_rows, typ_rows, wtab):
        kfn = pl.kernel(
            body,
            out_type=jax.ShapeDtypeStruct((NC, acc_n, F), jnp.float32),
            mesh=mesh,
            scratch_types=[
                pltpu.VMEM(wtab.shape, jnp.float32),
                pltpu.VMEM((ib, 128), jnp.int32),
                pltpu.VMEM((ib, 128), jnp.int32),
                pltpu.VMEM((ib, 128), jnp.int32),
                pltpu.VMEM((128, F), jnp.float32),
                pltpu.VMEM((128, F), jnp.float32),
                pltpu.VMEM((128, F), jnp.float32),
                pltpu.VMEM((128, F), jnp.float32),
                pltpu.VMEM((128, F), jnp.float32),
                pltpu.VMEM((128, F), jnp.float32),
                pltpu.VMEM_SHARED((acc_n, F), jnp.float32),
                pltpu.SemaphoreType.DMA,
                pltpu.SemaphoreType.DMA,
                pltpu.SemaphoreType.DMA,
                pltpu.SemaphoreType.DMA,
                pltpu.SemaphoreType.DMA,
                pltpu.SemaphoreType.DMA,
            ],
            compiler_params=pltpu.CompilerParams(
                needs_layout_passes=False, use_tc_tiling_on_sc=False),
        )
        return kfn(h16, src_rows, dst_rows, typ_rows, wtab)

    return run


def _make_tc_epilogue(n_nodes, out_dim, cnt_col, final):
    bn = 2000
    grid = n_nodes // bn

    def body(agg_ref, h_ref, root_ref, bias_ref, out_ref):
        aggs = agg_ref[0] + agg_ref[1]
        if cnt_col is not None:
            cnt = jnp.maximum(aggs[:, cnt_col:cnt_col + 1], 1.0)
            aggs = aggs / cnt
        dense = jnp.dot(h_ref[...], root_ref[...],
                        preferred_element_type=jnp.float32)
        t = aggs + dense + bias_ref[...]
        if final:
            t4 = t[:, :out_dim]
            m = jnp.max(t4, axis=1, keepdims=True)
            z = t4 - m
            lse = jnp.log(jnp.sum(jnp.exp(z), axis=1, keepdims=True))
            out_ref[...] = z - lse
        else:
            t = jnp.maximum(t, 0.0)
            mask = lax.broadcasted_iota(jnp.int32, t.shape, 1) < out_dim
            out_ref[...] = jnp.where(mask, t, 0.0)

    out_w = out_dim if final else F

    def run(agg, h16, rootp, biasp):
        return pl.pallas_call(
            body,
            grid=(grid,),
            in_specs=[
                pl.BlockSpec((NC, bn, F), lambda i: (0, i, 0)),
                pl.BlockSpec((bn, F), lambda i: (i, 0)),
                pl.BlockSpec((F, F), lambda i: (0, 0)),
                pl.BlockSpec((1, F), lambda i: (0, 0)),
            ],
            out_specs=pl.BlockSpec((bn, out_w), lambda i: (i, 0)),
            out_shape=jax.ShapeDtypeStruct((n_nodes, out_w), jnp.float32),
        )(agg, h16, rootp, biasp)

    return run


def _pad_mat(m):
    return jnp.pad(m, ((0, F - m.shape[0]), (0, F - m.shape[1])))


def kernel(x, edge_index, edge_type, W1, root1, b1, W2, root2, b2,
           W3, root3, b3):
    n = x.shape[0]
    e = edge_type.shape[0]
    r = W1.shape[0]

    erows = _round_up(e, 128 * NW * 8) // 128
    epad = erows * 128 - e

    src = jnp.concatenate([edge_index[0],
                           jnp.zeros((epad,), jnp.int32)]).reshape(erows, 128)
    dst = jnp.concatenate([edge_index[1],
                           jnp.full((epad,), n, jnp.int32)]).reshape(erows, 128)
    typ = jnp.concatenate([edge_type,
                           jnp.zeros((epad,), jnp.int32)]).reshape(erows, 128)

    x16 = jnp.pad(x, ((0, 0), (0, F - x.shape[1])))
    w1t = W1.reshape(r, -1)
    w2t = W2.reshape(r, -1)
    w3t = W3.reshape(r, -1)

    sc1 = _make_sc_layer(n, erows, 4, 2, 8, w1t.shape[1], count_col=8)
    sc2 = _make_sc_layer(n, erows, 8, 4, 12, w2t.shape[1], count_col=None)
    sc3 = _make_sc_layer(n, erows, 12, 2, 4, w3t.shape[1], count_col=4)
    tc1 = _make_tc_epilogue(n, 8, cnt_col=8, final=False)
    tc2 = _make_tc_epilogue(n, 12, cnt_col=None, final=False)
    tc3 = _make_tc_epilogue(n, 4, cnt_col=4, final=True)

    agg1 = sc1(x16, src, dst, typ, w1t)
    h1 = tc1(agg1, x16, _pad_mat(root1), jnp.pad(b1, (0, F - 8))[None, :])
    agg2 = sc2(h1, src, dst, typ, w2t)
    h2 = tc2(agg2, h1, _pad_mat(root2), jnp.pad(b2, (0, F - 12))[None, :])
    agg3 = sc3(h2, src, dst, typ, w3t)
    out = tc3(agg3, h2, _pad_mat(root3), jnp.pad(b3, (0, F - 4))[None, :])
    return out


# speedup vs baseline: 1.2031x; 1.2031x over previous
import functools

import jax
import jax.numpy as jnp
from jax import lax
from jax.experimental import pallas as pl
from jax.experimental.pallas import tpu as pltpu
from jax.experimental.pallas import tpu_sc as plsc

F = 16
LANES = 16
NC = 2
NS = 16
NW = NC * NS


def _round_up(v, m):
    return (v + m - 1) // m * m


def _largest_divisor_leq(n, cap):
    for d in range(min(cap, n), 0, -1):
        if n % d == 0:
            return d
    return 1


def _make_sc_layer(n_nodes, units, in_dim, n_blocks, out_dim, ws, count_col):
    ibk = in_dim // n_blocks
    obk = out_dim // n_blocks
    acc_n = _round_up(n_nodes + 1, 2048)
    zps = acc_n // NS
    nz = zps // 128
    ob = 128
    no = zps // ob

    mesh = plsc.VectorSubcoreMesh(core_axis_name="c", subcore_axis_name="s")

    def body(h_hbm, ei_hbm, typ_hbm, wtab_hbm, out_hbm,
             wtab_v, eib, typb, rows0, rows1, msg0, msg1, acc_sh,
             sem0, sem1, sem2, sem3):
        c = lax.axis_index("c")
        s = lax.axis_index("s")
        wid = s * NC + c

        pltpu.sync_copy(wtab_hbm, wtab_v)

        zero16 = jnp.zeros((LANES,), jnp.float32)
        for i in range(128):
            msg0[i, :] = zero16
            msg1[i, :] = zero16

        def zero_body(k, _):
            pltpu.sync_copy(msg0, acc_sh.at[pl.ds(s * zps + k * 128, 128)])
            return 0
        lax.fori_loop(0, nz, zero_body, 0)
        plsc.subcore_barrier()

        lanes = lax.iota(jnp.int32, LANES)
        ones16 = jnp.ones((LANES,), jnp.float32)

        u0 = (wid * units) // NW
        u1 = ((wid + 1) * units) // NW

        def compute_row(rbuf, mbuf, j):
            for g in range(128 // LANES):
                rowi = g * LANES + lanes
                tvec = typb[pl.ds(j * 128 + g * LANES, LANES)]
                hc = [plsc.load_gather(
                          rbuf, [rowi, jnp.full((LANES,), i, jnp.int32)])
                      for i in range(in_dim)]
                wc = [plsc.load_gather(
                          wtab_v, [tvec, jnp.full((LANES,), k, jnp.int32)])
                      for k in range(n_blocks * ibk * obk)]
                for o in range(out_dim):
                    b = o // obk
                    oo = o % obk
                    acc = hc[b * ibk] * wc[(b * ibk) * obk + oo]
                    for i in range(1, ibk):
                        acc = acc + hc[b * ibk + i] * wc[(b * ibk + i) * obk + oo]
                    plsc.store_scatter(
                        mbuf, [rowi, jnp.full((LANES,), o, jnp.int32)], acc)
                if count_col is not None:
                    plsc.store_scatter(
                        mbuf,
                        [rowi, jnp.full((LANES,), count_col, jnp.int32)],
                        ones16)

        def src_at(j):
            return eib.at[0, pl.ds(j * 128, 128)]

        def dst_at(j):
            return eib.at[1, pl.ds(j * 128, 128)]

        def do_block(base_u, ib):
            base_e = base_u * 1024
            pltpu.sync_copy(ei_hbm.at[:, pl.ds(base_e, ib * 128)],
                            eib.at[:, pl.ds(0, ib * 128)])
            pltpu.sync_copy(typ_hbm.at[pl.ds(base_e, ib * 128)],
                            typb.at[pl.ds(0, ib * 128)])

            pltpu.async_copy(h_hbm.at[src_at(0)], rows0, sem0)

            def pair_body(jp, _):
                j0 = 2 * jp
                pltpu.async_copy(h_hbm.at[src_at(j0 + 1)], rows1, sem1)
                pltpu.make_async_copy(h_hbm.at[src_at(j0)], rows0,
                                      sem0).wait()

                @pl.when(jp > 0)
                def _():
                    pltpu.make_async_copy(msg0, acc_sh.at[dst_at(j0)],
                                          sem2).wait()
                compute_row(rows0, msg0, j0)
                pltpu.async_copy(msg0, acc_sh.at[dst_at(j0)], sem2,
                                 add=True)

                @pl.when(jp + 1 < ib // 2)
                def _():
                    pltpu.async_copy(h_hbm.at[src_at(j0 + 2)], rows0, sem0)
                pltpu.make_async_copy(h_hbm.at[src_at(j0 + 1)], rows1,
                                      sem1).wait()

                @pl.when(jp > 0)
                def _():
                    pltpu.make_async_copy(msg1, acc_sh.at[dst_at(j0 + 1)],
                                          sem3).wait()
                compute_row(rows1, msg1, j0 + 1)
                pltpu.async_copy(msg1, acc_sh.at[dst_at(j0 + 1)], sem3,
                                 add=True)
                return 0
            lax.fori_loop(0, ib // 2, pair_body, 0)
            pltpu.make_async_copy(msg0, acc_sh.at[dst_at(0)], sem2).wait()
            pltpu.make_async_copy(msg1, acc_sh.at[dst_at(1)], sem3).wait()

        nb4 = (u1 - u0) // 4

        def blk32(ci, _):
            do_block(u0 + 4 * ci, 32)
            return 0
        lax.fori_loop(0, nb4, blk32, 0)

        def blk8(ri, _):
            do_block(u0 + 4 * nb4 + ri, 8)
            return 0
        lax.fori_loop(0, (u1 - u0) - 4 * nb4, blk8, 0)
        plsc.subcore_barrier()

        def out_body(k, _):
            r = s * zps + k * ob
            pltpu.sync_copy(acc_sh.at[pl.ds(r, ob)], rows0)
            pltpu.sync_copy(rows0, out_hbm.at[c, pl.ds(r, ob)])
            return 0
        lax.fori_loop(0, no, out_body, 0)

    def run(h16, ei, typ, wtab):
        kfn = pl.kernel(
            body,
            out_type=jax.ShapeDtypeStruct((NC, acc_n, F), jnp.float32),
            mesh=mesh,
            scratch_types=[
                pltpu.VMEM(wtab.shape, jnp.float32),
                pltpu.VMEM((2, 32 * 128), jnp.int32),
                pltpu.VMEM((32 * 128,), jnp.int32),
                pltpu.VMEM((128, F), jnp.float32),
                pltpu.VMEM((128, F), jnp.float32),
                pltpu.VMEM((128, F), jnp.float32),
                pltpu.VMEM((128, F), jnp.float32),
                pltpu.VMEM_SHARED((acc_n, F), jnp.float32),
                pltpu.SemaphoreType.DMA,
                pltpu.SemaphoreType.DMA,
                pltpu.SemaphoreType.DMA,
                pltpu.SemaphoreType.DMA,
            ],
            compiler_params=pltpu.CompilerParams(
                needs_layout_passes=False, use_tc_tiling_on_sc=False),
        )
        return kfn(h16, ei, typ, wtab)

    return run


def _make_tc_epilogue(n_nodes, out_dim, cnt_col, final):
    bn = 10000
    grid = n_nodes // bn

    def body(agg_ref, h_ref, root_ref, bias_ref, out_ref):
        aggs = agg_ref[0] + agg_ref[1]
        if cnt_col is not None:
            cnt = jnp.maximum(aggs[:, cnt_col:cnt_col + 1], 1.0)
            aggs = aggs / cnt
        dense = jnp.dot(h_ref[...], root_ref[...],
                        preferred_element_type=jnp.float32)
        t = aggs + dense + bias_ref[...]
        if final:
            t4 = t[:, :out_dim]
            m = jnp.max(t4, axis=1, keepdims=True)
            z = t4 - m
            lse = jnp.log(jnp.sum(jnp.exp(z), axis=1, keepdims=True))
            out_ref[...] = z - lse
        else:
            t = jnp.maximum(t, 0.0)
            mask = lax.broadcasted_iota(jnp.int32, t.shape, 1) < out_dim
            out_ref[...] = jnp.where(mask, t, 0.0)

    out_w = out_dim if final else F

    def run(agg, h16, rootp, biasp):
        return pl.pallas_call(
            body,
            grid=(grid,),
            in_specs=[
                pl.BlockSpec((NC, bn, F), lambda i: (0, i, 0)),
                pl.BlockSpec((bn, F), lambda i: (i, 0)),
                pl.BlockSpec((F, F), lambda i: (0, 0)),
                pl.BlockSpec((1, F), lambda i: (0, 0)),
            ],
            out_specs=pl.BlockSpec((bn, out_w), lambda i: (i, 0)),
            out_shape=jax.ShapeDtypeStruct((n_nodes, out_w), jnp.float32),
        )(agg, h16, rootp, biasp)

    return run


def _pad_mat(m):
    return jnp.pad(m, ((0, F - m.shape[0]), (0, F - m.shape[1])))


def kernel(x, edge_index, edge_type, W1, root1, b1, W2, root2, b2,
           W3, root3, b3):
    n = x.shape[0]
    e = edge_type.shape[0]
    r = W1.shape[0]

    epad = _round_up(e, 1024) - e
    if epad:
        ei = jnp.concatenate(
            [edge_index,
             jnp.concatenate([jnp.zeros((1, epad), jnp.int32),
                              jnp.full((1, epad), n, jnp.int32)])], axis=1)
        typ = jnp.concatenate([edge_type, jnp.zeros((epad,), jnp.int32)])
    else:
        ei = edge_index
        typ = edge_type
    units = (e + epad) // 1024

    x16 = jnp.pad(x, ((0, 0), (0, F - x.shape[1])))
    w1t = W1.reshape(r, -1)
    w2t = W2.reshape(r, -1)
    w3t = W3.reshape(r, -1)

    sc1 = _make_sc_layer(n, units, 4, 2, 8, w1t.shape[1], count_col=8)
    sc2 = _make_sc_layer(n, units, 8, 4, 12, w2t.shape[1], count_col=None)
    sc3 = _make_sc_layer(n, units, 12, 2, 4, w3t.shape[1], count_col=4)
    tc1 = _make_tc_epilogue(n, 8, cnt_col=8, final=False)
    tc2 = _make_tc_epilogue(n, 12, cnt_col=None, final=False)
    tc3 = _make_tc_epilogue(n, 4, cnt_col=4, final=True)

    agg1 = sc1(x16, ei, typ, w1t)
    h1 = tc1(agg1, x16, _pad_mat(root1), jnp.pad(b1, (0, F - 8))[None, :])
    agg2 = sc2(h1, ei, typ, w2t)
    h2 = tc2(agg2, h1, _pad_mat(root2), jnp.pad(b2, (0, F - 12))[None, :])
    agg3 = sc3(h2, ei, typ, w3t)
    out = tc3(agg3, h2, _pad_mat(root3), jnp.pad(b3, (0, F - 4))[None, :])
    return out
